# baseline (speedup 1.0000x reference)
import functools

import jax
import jax.numpy as jnp
from jax import lax
from jax.experimental import pallas as pl
from jax.experimental.pallas import tpu as pltpu
from jax.experimental.pallas import tpu_sc as plsc

N_NODES = 10000
D = 128
NC, NS = 2, 16
B = 128
CPP = 40
NP0, NP1 = 4, 0
NPT = NP0 + NP1
ACC_ROWS = 10240
ROWS_PER_TILE = ACC_ROWS // NS


def _sc_aggregate(x, src2, dst2, zeros):
    mesh = plsc.VectorSubcoreMesh(core_axis_name="c", subcore_axis_name="s")

    @functools.partial(
        pl.kernel,
        out_type=jax.ShapeDtypeStruct((ACC_ROWS, D), jnp.float32),
        mesh=mesh,
        scratch_types=[
            pltpu.VMEM((CPP, B), jnp.int32),
            pltpu.VMEM((CPP, B), jnp.int32),
            pltpu.VMEM((B, D), jnp.float32),
            pltpu.VMEM((B, D), jnp.float32),
            pltpu.VMEM_SHARED((ACC_ROWS, D), jnp.float32),
            pltpu.SemaphoreType.DMA,
            pltpu.SemaphoreType.DMA,
        ],
    )
    def k(x_hbm, src_hbm, dst_hbm, zeros_hbm, out_hbm, src_v, dst_v, rows0,
          rows1, acc, sem0, sem1):
        c = lax.axis_index("c")
        s = lax.axis_index("s")

        with jax.named_scope("zinit"):
            pltpu.sync_copy(zeros_hbm, rows0)
            for r in range(ROWS_PER_TILE // B):
                pltpu.sync_copy(
                    rows0, acc.at[pl.ds(s * ROWS_PER_TILE + r * B, B)])
            plsc.subcore_barrier()

        npass = lax.select(c == 0, NP0, NP1)

        def do_pass(p, carry):
            row0 = (s * NPT + p) * CPP
            pltpu.sync_copy(src_hbm.at[pl.ds(row0, CPP)], src_v)
            pltpu.sync_copy(dst_hbm.at[pl.ds(row0, CPP)], dst_v)

            pltpu.async_copy(x_hbm.at[src_v.at[0]], rows0, sem0)
            pltpu.async_copy(x_hbm.at[src_v.at[1]], rows1, sem1)

            def body(i, cr):
                j0 = 2 * i
                j1 = j0 + 1
                pltpu.make_async_copy(x_hbm.at[src_v.at[j0]], rows0, sem0).wait()
                pltpu.sync_copy(rows0, acc.at[dst_v.at[j0]], add=True)
                pltpu.async_copy(x_hbm.at[src_v.at[j0 + 2]], rows0, sem0)
                pltpu.make_async_copy(x_hbm.at[src_v.at[j1]], rows1, sem1).wait()
                pltpu.sync_copy(rows1, acc.at[dst_v.at[j1]], add=True)
                pltpu.async_copy(x_hbm.at[src_v.at[j1 + 2]], rows1, sem1)
                return cr

            lax.fori_loop(0, CPP // 2 - 1, body, 0)
            jt = CPP - 2
            pltpu.make_async_copy(x_hbm.at[src_v.at[jt]], rows0, sem0).wait()
            pltpu.sync_copy(rows0, acc.at[dst_v.at[jt]], add=True)
            pltpu.make_async_copy(x_hbm.at[src_v.at[jt + 1]], rows1, sem1).wait()
            pltpu.sync_copy(rows1, acc.at[dst_v.at[jt + 1]], add=True)
            return carry

        with jax.named_scope("mainloop"):
            lax.fori_loop(0, npass, do_pass, 0)
            plsc.subcore_barrier()

        with jax.named_scope("export"):

            @pl.when(c == 0)
            def _():
                pltpu.sync_copy(
                    acc.at[pl.ds(s * ROWS_PER_TILE, ROWS_PER_TILE)],
                    out_hbm.at[pl.ds(s * ROWS_PER_TILE, ROWS_PER_TILE)],
                )

    return k(x, src2, dst2, zeros)


def kernel(x, edge_index):
    src = edge_index[0].astype(jnp.int32)
    dst = edge_index[1].astype(jnp.int32)
    e = src.shape[0]
    g = NS * NPT * CPP * B
    assert e <= g, (e, g)
    pad = g - e
    if pad:
        src = jnp.concatenate([src, jnp.zeros((pad,), jnp.int32)])
        dst = jnp.concatenate([dst, jnp.full((pad,), N_NODES, jnp.int32)])
    src2 = src.reshape(-1, B)
    dst2 = dst.reshape(-1, B)
    zeros = jnp.zeros((B, D), jnp.float32)
    out = _sc_aggregate(x, src2, dst2, zeros)
    return out[:N_NODES]


# speedup vs baseline: 1.4387x; 1.4387x over previous
import functools

import jax
import jax.numpy as jnp
from jax import lax
from jax.experimental import pallas as pl
from jax.experimental.pallas import tpu as pltpu
from jax.experimental.pallas import tpu_sc as plsc

N_NODES = 10000
D = 128
NC, NS = 2, 16
NW = NC * NS
B = 128
ACC_ROWS = 10240
ROWS_PER_TILE = ACC_ROWS // NS


def _sc_partials(x, src3, dst3, zeros):
    nchunks = src3.shape[1]
    mesh = plsc.VectorSubcoreMesh(core_axis_name="c", subcore_axis_name="s")

    @functools.partial(
        pl.kernel,
        out_type=jax.ShapeDtypeStruct((NC, ACC_ROWS, D), jnp.float32),
        mesh=mesh,
        scratch_types=[
            pltpu.VMEM((nchunks, B), jnp.int32),
            pltpu.VMEM((nchunks, B), jnp.int32),
            pltpu.VMEM((B, D), jnp.float32),
            pltpu.VMEM_SHARED((ACC_ROWS, D), jnp.float32),
            pltpu.SemaphoreType.DMA,
        ],
    )
    def k(x_hbm, src_hbm, dst_hbm, zeros_hbm, out_hbm, src_v, dst_v, rows_v,
          acc, sem):
        c = lax.axis_index("c")
        s = lax.axis_index("s")
        w = s * NC + c

        with jax.named_scope("zinit"):
            pltpu.sync_copy(
                zeros_hbm, acc.at[pl.ds(s * ROWS_PER_TILE, ROWS_PER_TILE)])
            plsc.subcore_barrier()

        with jax.named_scope("mainloop"):
            pltpu.sync_copy(src_hbm.at[w], src_v)
            pltpu.sync_copy(dst_hbm.at[w], dst_v)

            def body(j, carry):
                pltpu.async_copy(x_hbm.at[src_v.at[j]], rows_v, sem).wait()
                pltpu.sync_copy(rows_v, acc.at[dst_v.at[j]], add=True)
                return carry

            lax.fori_loop(0, nchunks, body, 0)
            plsc.subcore_barrier()

        with jax.named_scope("export"):
            pltpu.sync_copy(
                acc.at[pl.ds(s * ROWS_PER_TILE, ROWS_PER_TILE)],
                out_hbm.at[c, pl.ds(s * ROWS_PER_TILE, ROWS_PER_TILE)],
            )

    return k(x, src3, dst3, zeros)


def _combine(partials):
    BLK = 1280

    def body(p_ref, o_ref):
        o_ref[...] = p_ref[0] + p_ref[1]

    out = pl.pallas_call(
        body,
        grid=(ACC_ROWS // BLK,),
        in_specs=[pl.BlockSpec((NC, BLK, D), lambda i: (0, i, 0))],
        out_specs=pl.BlockSpec((BLK, D), lambda i: (i, 0)),
        out_shape=jax.ShapeDtypeStruct((ACC_ROWS, D), jnp.float32),
    )(partials)
    return out[:N_NODES]


def kernel(x, edge_index):
    src = edge_index[0].astype(jnp.int32)
    dst = edge_index[1].astype(jnp.int32)
    e = src.shape[0]
    g = NW * B
    e_pad = ((e + g - 1) // g) * g
    pad = e_pad - e
    if pad:
        src = jnp.concatenate([src, jnp.zeros((pad,), jnp.int32)])
        dst = jnp.concatenate([dst, jnp.full((pad,), N_NODES, jnp.int32)])
    src3 = src.reshape(NW, -1, B)
    dst3 = dst.reshape(NW, -1, B)
    zeros = jnp.zeros((ROWS_PER_TILE, D), jnp.float32)
    partials = _sc_partials(x, src3, dst3, zeros)
    return _combine(partials)
